# baseline (speedup 1.0000x reference)
import jax
import jax.numpy as jnp
from jax.experimental import pallas as pl
from jax.experimental.pallas import tpu as pltpu

N = 16384
BM = 256
NBLK = N // BM


def _body(m_ref, v_ref, r_ref, out_ref, sq_ref, mx_ref):
    i = pl.program_id(0)
    out = jnp.dot(m_ref[...], v_ref[...],
                  preferred_element_type=jnp.float32)[:, 0]
    out_ref[...] = out
    err = out - r_ref[...]
    sq = jnp.sum(err * err).reshape(1, 1)
    mx = jnp.max(jnp.abs(err)).reshape(1, 1)

    @pl.when(i == 0)
    def _init():
        sq_ref[...] = sq
        mx_ref[...] = mx

    @pl.when(i > 0)
    def _acc():
        sq_ref[...] += sq
        mx_ref[...] = jnp.maximum(mx_ref[...], mx)


def kernel(matrix, vector, ref):
    out, sq, mx = pl.pallas_call(
        _body,
        grid=(NBLK,),
        in_specs=[
            pl.BlockSpec((BM, N), lambda i: (i, 0)),
            pl.BlockSpec((N, 1), lambda i: (0, 0)),
            pl.BlockSpec((BM,), lambda i: (i,)),
        ],
        out_specs=[
            pl.BlockSpec((BM,), lambda i: (i,)),
            pl.BlockSpec((1, 1), lambda i: (0, 0)),
            pl.BlockSpec((1, 1), lambda i: (0, 0)),
        ],
        out_shape=[
            jax.ShapeDtypeStruct((N,), jnp.float32),
            jax.ShapeDtypeStruct((1, 1), jnp.float32),
            jax.ShapeDtypeStruct((1, 1), jnp.float32),
        ],
    )(matrix, vector, ref)
    loss = sq[0, 0] / jnp.float32(N)
    return (loss, out, ref, mx[0, 0])
